# baseline (speedup 1.0000x reference)
import functools

import jax
import jax.numpy as jnp
from jax import lax
from jax.experimental import pallas as pl
from jax.experimental.pallas import tpu as pltpu
from jax.experimental.pallas import tpu_sc as plsc

N = 10000
E = 320000
D = 128
H = 128
DH = D // 2
NC = 2
NS = 16
EPT = E // NS
CHUNK = 32
NCHUNK = EPT // CHUNK
NBUFG = 2
NBUFS = 1
K = 1
RPT = 624
REM = N - NS * RPT
ZROWS = 208
DEGR = N // 16


def _sc_agg_body(xs_hbm, src_hbm, dst_hbm, ew_hbm, eye_hbm,
                 agg_out, deg_out,
                 src_v, dst_v, ew_v, rows_v, sc_v, id_v, deg_l,
                 zbuf,
                 acc_sh, gsem, ssem):
    cid = lax.axis_index("c")
    sid = lax.axis_index("s")
    wid = cid * NS + sid

    pltpu.sync_copy(src_hbm.at[sid], src_v)
    pltpu.sync_copy(dst_hbm.at[sid], dst_v)
    pltpu.sync_copy(ew_hbm.at[sid], ew_v)

    zero16 = jnp.zeros((16,), jnp.float32)

    def _zrow(i, carry):
        for k in range(DH // 16):
            zbuf[i, pl.ds(k * 16, 16)] = zero16
        return carry
    lax.fori_loop(0, ZROWS, _zrow, 0)

    pltpu.sync_copy(eye_hbm, id_v)

    for kk in range(RPT // ZROWS):
        sl = pl.ds(sid * RPT + kk * ZROWS, ZROWS)
        pltpu.sync_copy(zbuf, acc_sh.at[sl])

    @pl.when(sid == 0)
    def _zero_rem():
        pltpu.sync_copy(zbuf.at[pl.ds(0, REM)], acc_sh.at[pl.ds(NS * RPT, REM)])

    zero16v = jnp.zeros((16,), jnp.float32)

    def _zdeg(i, carry):
        deg_l[i, pl.ds(0, 16)] = zero16v
        return carry
    lax.fori_loop(0, DEGR, _zdeg, 0)

    plsc.subcore_barrier()

    def _gather(j, b):
        return pltpu.make_async_copy(xs_hbm.at[cid].at[src_v.at[j]],
                                     rows_v.at[b], gsem)

    def _accsc(j, b):
        return pltpu.make_async_copy(sc_v.at[b],
                                     acc_sh.at[dst_v.at[j]], ssem)

    def _step(t, carry):
        @pl.when((t >= K + NBUFS) & (t < NCHUNK + K + NBUFS))
        def _drain():
            jd = t - K - NBUFS
            _accsc(jd, lax.rem(jd, NBUFS)).wait()

        @pl.when(t < NCHUNK)
        def _gfire():
            _gather(t, lax.rem(t, NBUFG)).start()

        @pl.when((t >= K) & (t < NCHUNK + K))
        def _consume():
            j = t - K
            bg = lax.rem(j, NBUFG)
            bs = lax.rem(j, NBUFS)
            _gather(j, bg).wait()

            def _scale(g2, c2):
                i0 = g2 * 16
                ewv = ew_v[j, pl.ds(i0, 16)]
                dv = dst_v[j, pl.ds(i0, 16)]
                ddiv = lax.shift_right_logical(dv, 4)
                dmod = lax.rem(dv, 16)
                for e in range(16):
                    wv = jnp.full((16,), ewv[e], jnp.float32)
                    for k in range(DH // 16):
                        sl = pl.ds(k * 16, 16)
                        sc_v[bs, i0 + e, sl] = rows_v[bg, i0 + e, sl] * wv
                    r = ddiv[e]
                    deg_l[r, pl.ds(0, 16)] = (deg_l[r, pl.ds(0, 16)]
                                              + id_v[dmod[e], pl.ds(0, 16)])
                return c2
            lax.fori_loop(0, CHUNK // 16, _scale, 0)

            _accsc(j, bs).start(add=True)

        return carry
    lax.fori_loop(0, NCHUNK + K + NBUFS, _step, 0)

    pltpu.sync_copy(deg_l, deg_out.at[wid])

    plsc.subcore_barrier()

    for kk in range(RPT // ZROWS):
        r0 = sid * RPT + kk * ZROWS
        pltpu.sync_copy(acc_sh.at[pl.ds(r0, ZROWS)],
                        agg_out.at[cid, pl.ds(r0, ZROWS)])

    @pl.when(sid == 0)
    def _copy_rem():
        pltpu.sync_copy(acc_sh.at[pl.ds(NS * RPT, REM)],
                        agg_out.at[cid, pl.ds(NS * RPT, REM)])


@functools.lru_cache(maxsize=1)
def _sc_aggregate_fn():
    return pl.kernel(
        _sc_agg_body,
        mesh=plsc.VectorSubcoreMesh(core_axis_name="c", subcore_axis_name="s"),
        compiler_params=pltpu.CompilerParams(use_tc_tiling_on_sc=False),
        out_type=[
            jax.ShapeDtypeStruct((NC, N, DH), jnp.float32),
            jax.ShapeDtypeStruct((NC * NS, DEGR, 16), jnp.float32),
        ],
        scratch_types=[
            pltpu.VMEM((NCHUNK, CHUNK), jnp.int32),
            pltpu.VMEM((NCHUNK, CHUNK), jnp.int32),
            pltpu.VMEM((NCHUNK, CHUNK), jnp.float32),
            pltpu.VMEM((NBUFG, CHUNK, DH), jnp.float32),
            pltpu.VMEM((NBUFS, CHUNK, DH), jnp.float32),
            pltpu.VMEM((16, 16), jnp.float32),
            pltpu.VMEM((DEGR, 16), jnp.float32),
            pltpu.VMEM((ZROWS, DH), jnp.float32),
            pltpu.VMEM_SHARED((N, DH), jnp.float32),
            pltpu.SemaphoreType.DMA,
            pltpu.SemaphoreType.DMA,
        ],
    )


def _tc_dense_body(p0, p1, dgp, x, h, c,
                   Wc, Wig, big, Whg, bhg, Wil, Whl, bl, wfc, bfc,
                   out, hn, cn):
    hp = jax.lax.Precision.HIGHEST
    agg = jnp.concatenate([p0[...], p1[...]], axis=1)
    deg = jnp.transpose(jnp.sum(dgp[0], axis=0, keepdims=True))
    aggn = agg / jnp.maximum(deg, 1.0)
    xb = x[...]
    aggw = jnp.dot(aggn, Wc[...], precision=hp)
    gi = jnp.dot(aggw, Wig[...], precision=hp) + big[...]
    gh = jnp.dot(xb, Whg[...], precision=hp) + bhg[...]
    r = jax.nn.sigmoid(gi[:, 0:H] + gh[:, 0:H])
    z = jax.nn.sigmoid(gi[:, H:2 * H] + gh[:, H:2 * H])
    n = jnp.tanh(gi[:, 2 * H:3 * H] + r * gh[:, 2 * H:3 * H])
    x_new = (1.0 - z) * n + z * xb
    gates = (jnp.dot(x_new, Wil[...], precision=hp)
             + jnp.dot(h[...], Whl[...], precision=hp) + bl[...])
    i_g = jax.nn.sigmoid(gates[:, 0:H])
    f_g = jax.nn.sigmoid(gates[:, H:2 * H])
    g_g = jnp.tanh(gates[:, 2 * H:3 * H])
    o_g = jax.nn.sigmoid(gates[:, 3 * H:4 * H])
    c_new = f_g * c[...] + i_g * g_g
    h_new = o_g * jnp.tanh(c_new)
    out[...] = jnp.sum(jax.nn.relu(h_new) * wfc[...], axis=1,
                       keepdims=True) + bfc[...]
    hn[...] = h_new
    cn[...] = c_new


def _tc_dense(p0, p1, dgp, x, h, c, Wc, Wig, big, Whg, bhg, Wil, Whl,
              bl, wfc, bfc, block=1000):
    nblk = N // block
    row = lambda i: (i, 0)
    fixed = lambda i: (0, 0)
    in_specs = [
        pl.BlockSpec((block, DH), row),
        pl.BlockSpec((block, DH), row),
        pl.BlockSpec((1, NC * NS, block), lambda i: (i, 0, 0)),
        pl.BlockSpec((block, D), row),
        pl.BlockSpec((block, H), row),
        pl.BlockSpec((block, H), row),
        pl.BlockSpec((D, D), fixed),
        pl.BlockSpec((D, 3 * D), fixed),
        pl.BlockSpec((1, 3 * D), fixed),
        pl.BlockSpec((D, 3 * D), fixed),
        pl.BlockSpec((1, 3 * D), fixed),
        pl.BlockSpec((D, 4 * H), fixed),
        pl.BlockSpec((H, 4 * H), fixed),
        pl.BlockSpec((1, 4 * H), fixed),
        pl.BlockSpec((1, H), fixed),
        pl.BlockSpec((1, 1), fixed),
    ]
    out_specs = [
        pl.BlockSpec((block, 1), row),
        pl.BlockSpec((block, H), row),
        pl.BlockSpec((block, H), row),
    ]
    out_shape = [
        jax.ShapeDtypeStruct((N, 1), jnp.float32),
        jax.ShapeDtypeStruct((N, H), jnp.float32),
        jax.ShapeDtypeStruct((N, H), jnp.float32),
    ]
    return pl.pallas_call(
        _tc_dense_body,
        grid=(nblk,),
        in_specs=in_specs,
        out_specs=out_specs,
        out_shape=out_shape,
    )(p0, p1, dgp, x, h, c, Wc, Wig, big, Whg, bhg, Wil, Whl, bl,
      wfc, bfc)


def kernel(x, edge_index, edge_weight, h, c, W_conv, W_ih_gru, W_hh_gru,
           b_ih_gru, b_hh_gru, W_ih_lstm, W_hh_lstm, b_ih_lstm, b_hh_lstm,
           W_fc, b_fc):
    src2 = edge_index[0].astype(jnp.int32).reshape(NS, NCHUNK, CHUNK)
    dst2 = edge_index[1].astype(jnp.int32).reshape(NS, NCHUNK, CHUNK)
    ew2 = edge_weight.reshape(NS, NCHUNK, CHUNK)
    xs = jnp.stack([x[:, :DH], x[:, DH:]])
    eye = jnp.eye(16, dtype=jnp.float32)

    agg_halves, deg = _sc_aggregate_fn()(xs, src2, dst2, ew2, eye)
    blk = 1000
    dgp = deg.reshape(NC * NS, N // blk, blk).transpose(1, 0, 2)

    out, hn, cn = _tc_dense(
        agg_halves[0], agg_halves[1], dgp, x, h, c,
        W_conv,
        W_ih_gru.T, b_ih_gru.reshape(1, 3 * D),
        W_hh_gru.T, b_hh_gru.reshape(1, 3 * D),
        W_ih_lstm.T, W_hh_lstm.T,
        (b_ih_lstm + b_hh_lstm).reshape(1, 4 * H),
        W_fc, b_fc.reshape(1, 1),
    )
    return out, hn, cn


# speedup vs baseline: 1.4675x; 1.4675x over previous
import functools

import jax
import jax.numpy as jnp
from jax import lax
from jax.experimental import pallas as pl
from jax.experimental.pallas import tpu as pltpu
from jax.experimental.pallas import tpu_sc as plsc

N = 10000
E = 320000
D = 128
H = 128
DH = D // 2
NC = 2
NS = 16
EPT = E // NS
CHUNK = 80
NCHUNK = EPT // CHUNK
NBUF = 2
K = 1
RPT = 624
REM = N - NS * RPT
ZROWS = 208
DEGW = 8


def _sc_agg_body(xs_hbm, src_hbm, dst_hbm, ew_hbm, aux_hbm, agg_out, deg_out,
                 src_v, dst_v, ew_v, rows_v, ones_v, zbuf,
                 acc_sh, deg_sh, gsem, ssem, dsem):
    cid = lax.axis_index("c")
    sid = lax.axis_index("s")

    pltpu.sync_copy(src_hbm.at[sid], src_v)
    pltpu.sync_copy(dst_hbm.at[sid], dst_v)
    pltpu.sync_copy(ew_hbm.at[sid], ew_v)

    zero16 = jnp.zeros((16,), jnp.float32)

    def _zrow(i, carry):
        for k in range(DH // 16):
            zbuf[i, pl.ds(k * 16, 16)] = zero16
        return carry
    lax.fori_loop(0, ZROWS, _zrow, 0)

    pltpu.sync_copy(aux_hbm.at[pl.ds(ZROWS, CHUNK)], ones_v)

    for kk in range(RPT // ZROWS):
        sl = pl.ds(sid * RPT + kk * ZROWS, ZROWS)
        pltpu.sync_copy(zbuf, acc_sh.at[sl])
        pltpu.sync_copy(aux_hbm.at[pl.ds(0, ZROWS)], deg_sh.at[sl])

    @pl.when(sid == 0)
    def _zero_rem():
        pltpu.sync_copy(zbuf.at[pl.ds(0, REM)], acc_sh.at[pl.ds(NS * RPT, REM)])
        pltpu.sync_copy(aux_hbm.at[pl.ds(0, REM)],
                        deg_sh.at[pl.ds(NS * RPT, REM)])

    plsc.subcore_barrier()

    def _gather(j, b):
        return pltpu.make_async_copy(xs_hbm.at[cid].at[src_v.at[j]],
                                     rows_v.at[b], gsem)

    def _accsc(j, b):
        return pltpu.make_async_copy(rows_v.at[b],
                                     acc_sh.at[dst_v.at[j]], ssem)

    def _degsc(j):
        return pltpu.make_async_copy(ones_v, deg_sh.at[dst_v.at[j]], dsem)

    def _mine(j):
        return (cid == 0) == (j < NCHUNK // 2)

    def _step(t, carry):
        @pl.when((t >= NBUF) & (t < NCHUNK + NBUF))
        def _drain_acc():
            jd = t - NBUF
            _accsc(jd, lax.rem(jd, NBUF)).wait()

        @pl.when((t >= NBUF) & (t < NCHUNK + NBUF) & _mine(t - NBUF))
        def _drain_deg():
            _degsc(t - NBUF).wait()

        @pl.when(t < NCHUNK)
        def _gfire():
            _gather(t, lax.rem(t, NBUF)).start()

        @pl.when((t >= K) & (t < NCHUNK + K))
        def _consume():
            j = t - K
            b = lax.rem(j, NBUF)
            _gather(j, b).wait()

            def _scale(g2, c2):
                i0 = g2 * 16
                ewv = ew_v[j, pl.ds(i0, 16)]
                for e in range(16):
                    wv = jnp.full((16,), ewv[e], jnp.float32)
                    for k in range(DH // 16):
                        sl = pl.ds(k * 16, 16)
                        rows_v[b, i0 + e, sl] = rows_v[b, i0 + e, sl] * wv
                return c2
            lax.fori_loop(0, CHUNK // 16, _scale, 0)

            _accsc(j, b).start(add=True)

            @pl.when(_mine(j))
            def _dfire():
                _degsc(j).start(add=True)

        return carry
    lax.fori_loop(0, NCHUNK + NBUF, _step, 0)

    plsc.subcore_barrier()

    for kk in range(RPT // ZROWS):
        r0 = sid * RPT + kk * ZROWS
        pltpu.sync_copy(acc_sh.at[pl.ds(r0, ZROWS)],
                        agg_out.at[cid, pl.ds(r0, ZROWS)])
        pltpu.sync_copy(deg_sh.at[pl.ds(r0, ZROWS)],
                        deg_out.at[cid, pl.ds(r0, ZROWS)])

    @pl.when(sid == 0)
    def _copy_rem():
        pltpu.sync_copy(acc_sh.at[pl.ds(NS * RPT, REM)],
                        agg_out.at[cid, pl.ds(NS * RPT, REM)])
        pltpu.sync_copy(deg_sh.at[pl.ds(NS * RPT, REM)],
                        deg_out.at[cid, pl.ds(NS * RPT, REM)])


@functools.lru_cache(maxsize=1)
def _sc_aggregate_fn():
    return pl.kernel(
        _sc_agg_body,
        mesh=plsc.VectorSubcoreMesh(core_axis_name="c", subcore_axis_name="s"),
        compiler_params=pltpu.CompilerParams(use_tc_tiling_on_sc=False),
        out_type=[
            jax.ShapeDtypeStruct((NC, N, DH), jnp.float32),
            jax.ShapeDtypeStruct((NC, N, DEGW), jnp.float32),
        ],
        scratch_types=[
            pltpu.VMEM((NCHUNK, CHUNK), jnp.int32),
            pltpu.VMEM((NCHUNK, CHUNK), jnp.int32),
            pltpu.VMEM((NCHUNK, CHUNK), jnp.float32),
            pltpu.VMEM((NBUF, CHUNK, DH), jnp.float32),
            pltpu.VMEM((CHUNK, DEGW), jnp.float32),
            pltpu.VMEM((ZROWS, DH), jnp.float32),
            pltpu.VMEM_SHARED((N, DH), jnp.float32),
            pltpu.VMEM_SHARED((N, DEGW), jnp.float32),
            pltpu.SemaphoreType.DMA,
            pltpu.SemaphoreType.DMA,
            pltpu.SemaphoreType.DMA,
        ],
    )


def _tc_dense_body(p0, p1, d0, d1, x, h, c,
                   Wc, Wig, big, Whg, bhg, Wil, Whl, bl, wfc, bfc,
                   out, hn, cn):
    hp = jax.lax.Precision.DEFAULT
    agg = jnp.concatenate([p0[...], p1[...]], axis=1)
    deg = (d0[...] + d1[...])[:, 0:1]
    aggn = agg / jnp.maximum(deg, 1.0)
    xb = x[...]
    aggw = jnp.dot(aggn, Wc[...], precision=hp)
    gi = jnp.dot(aggw, Wig[...], precision=hp) + big[...]
    gh = jnp.dot(xb, Whg[...], precision=hp) + bhg[...]
    r = jax.nn.sigmoid(gi[:, 0:H] + gh[:, 0:H])
    z = jax.nn.sigmoid(gi[:, H:2 * H] + gh[:, H:2 * H])
    n = jnp.tanh(gi[:, 2 * H:3 * H] + r * gh[:, 2 * H:3 * H])
    x_new = (1.0 - z) * n + z * xb
    gates = (jnp.dot(x_new, Wil[...], precision=hp)
             + jnp.dot(h[...], Whl[...], precision=hp) + bl[...])
    i_g = jax.nn.sigmoid(gates[:, 0:H])
    f_g = jax.nn.sigmoid(gates[:, H:2 * H])
    g_g = jnp.tanh(gates[:, 2 * H:3 * H])
    o_g = jax.nn.sigmoid(gates[:, 3 * H:4 * H])
    c_new = f_g * c[...] + i_g * g_g
    h_new = o_g * jnp.tanh(c_new)
    out[...] = jnp.sum(jax.nn.relu(h_new) * wfc[...], axis=1,
                       keepdims=True) + bfc[...]
    hn[...] = h_new
    cn[...] = c_new


def _tc_dense(p0, p1, d0, d1, x, h, c, Wc, Wig, big, Whg, bhg, Wil, Whl,
              bl, wfc, bfc, block=1000):
    nblk = N // block
    row = lambda i: (i, 0)
    fixed = lambda i: (0, 0)
    in_specs = [
        pl.BlockSpec((block, DH), row),
        pl.BlockSpec((block, DH), row),
        pl.BlockSpec((block, DEGW), row),
        pl.BlockSpec((block, DEGW), row),
        pl.BlockSpec((block, D), row),
        pl.BlockSpec((block, H), row),
        pl.BlockSpec((block, H), row),
        pl.BlockSpec((D, D), fixed),
        pl.BlockSpec((D, 3 * D), fixed),
        pl.BlockSpec((1, 3 * D), fixed),
        pl.BlockSpec((D, 3 * D), fixed),
        pl.BlockSpec((1, 3 * D), fixed),
        pl.BlockSpec((D, 4 * H), fixed),
        pl.BlockSpec((H, 4 * H), fixed),
        pl.BlockSpec((1, 4 * H), fixed),
        pl.BlockSpec((1, H), fixed),
        pl.BlockSpec((1, 1), fixed),
    ]
    out_specs = [
        pl.BlockSpec((block, 1), row),
        pl.BlockSpec((block, H), row),
        pl.BlockSpec((block, H), row),
    ]
    out_shape = [
        jax.ShapeDtypeStruct((N, 1), jnp.float32),
        jax.ShapeDtypeStruct((N, H), jnp.float32),
        jax.ShapeDtypeStruct((N, H), jnp.float32),
    ]
    return pl.pallas_call(
        _tc_dense_body,
        grid=(nblk,),
        in_specs=in_specs,
        out_specs=out_specs,
        out_shape=out_shape,
    )(p0, p1, d0, d1, x, h, c, Wc, Wig, big, Whg, bhg, Wil, Whl, bl,
      wfc, bfc)


def kernel(x, edge_index, edge_weight, h, c, W_conv, W_ih_gru, W_hh_gru,
           b_ih_gru, b_hh_gru, W_ih_lstm, W_hh_lstm, b_ih_lstm, b_hh_lstm,
           W_fc, b_fc):
    src2 = edge_index[0].astype(jnp.int32).reshape(NS, NCHUNK, CHUNK)
    dst2 = edge_index[1].astype(jnp.int32).reshape(NS, NCHUNK, CHUNK)
    ew2 = edge_weight.reshape(NS, NCHUNK, CHUNK)
    xs = jnp.stack([x[:, :DH], x[:, DH:]])
    aux = jnp.concatenate([jnp.zeros((ZROWS, DEGW), jnp.float32),
                           jnp.ones((CHUNK, DEGW), jnp.float32)])

    agg_halves, deg = _sc_aggregate_fn()(xs, src2, dst2, ew2, aux)

    out, hn, cn = _tc_dense(
        agg_halves[0], agg_halves[1], deg[0], deg[1], x, h, c,
        W_conv,
        W_ih_gru.T, b_ih_gru.reshape(1, 3 * D),
        W_hh_gru.T, b_hh_gru.reshape(1, 3 * D),
        W_ih_lstm.T, W_hh_lstm.T,
        (b_ih_lstm + b_hh_lstm).reshape(1, 4 * H),
        W_fc, b_fc.reshape(1, 1),
    )
    return out, hn, cn
